# Initial kernel scaffold; baseline (speedup 1.0000x reference)
import math
import numpy as np
import jax
import jax.numpy as jnp
from jax import lax
from jax.experimental import pallas as pl
from jax.experimental.pallas import tpu as pltpu
from jax.experimental.pallas import tpu_sc as plsc

B, L = 32, 100
D = 128
LAYERS = 2
TOPK = 4
PN = 5000

NC, NS = 2, 16
NW = NC * NS
TPW = L
TPW_PAD = 104
CH = 8

_MESH = plsc.VectorSubcoreMesh(
    core_axis_name="c", subcore_axis_name="s", num_cores=NC, num_subcores=NS
)

_F32 = jnp.float32
_BIG = 3.0e38


def _wid():
    return lax.axis_index("c") * NS + lax.axis_index("s")


def _acc_add(acc, buf):
    def body(j, _):
        for u in range(8):
            sl = pl.ds(u * 16, 16)
            acc[j, sl] = acc[j, sl] + buf[j, sl]
        return 0
    lax.fori_loop(0, TPW, body, 0)


def _emb_body(uidx, pidx, cidx, tidx, didx,
              ue, pe, ce, te, de, uec, tec, dec,
              out_main, out_cat,
              iu, ip, ic, it, idw, acc, buf, sem):
    w = _wid()
    pltpu.sync_copy(uidx.at[w], iu)
    pltpu.sync_copy(pidx.at[w], ip)
    pltpu.sync_copy(cidx.at[w], ic)
    pltpu.sync_copy(tidx.at[w], it)
    pltpu.sync_copy(didx.at[w], idw)

    for first, rest, out in (
        ((ue, iu), ((pe, ip), (te, it), (de, idw)), out_main),
        ((uec, iu), ((ce, ic), (tec, it), (dec, idw)), out_cat),
    ):
        pltpu.async_copy(first[0].at[first[1].at[pl.ds(0, TPW)]], acc, sem).wait()
        for tbl, iv in rest:
            pltpu.async_copy(tbl.at[iv.at[pl.ds(0, TPW)]], buf, sem).wait()
            _acc_add(acc, buf)
        pltpu.sync_copy(acc, out.at[w])


def _emb_call(uidx, pidx, cidx, tidx, didx, tables):
    kfn = pl.kernel(
        _emb_body,
        out_type=(
            jax.ShapeDtypeStruct((NW, TPW, D), _F32),
            jax.ShapeDtypeStruct((NW, TPW, D), _F32),
        ),
        mesh=_MESH,
        scratch_types=(
            pltpu.VMEM((TPW_PAD,), jnp.int32),
            pltpu.VMEM((TPW_PAD,), jnp.int32),
            pltpu.VMEM((TPW_PAD,), jnp.int32),
            pltpu.VMEM((TPW_PAD,), jnp.int32),
            pltpu.VMEM((TPW_PAD,), jnp.int32),
            pltpu.VMEM((TPW, D), _F32),
            pltpu.VMEM((TPW, D), _F32),
            pltpu.SemaphoreType.DMA,
        ),
    )
    return kfn(uidx, pidx, cidx, tidx, didx, *tables)


def _minmax_body(dm, pidx, out_mn, out_mx, iv, buf, stg, sem):
    w = _wid()
    pltpu.sync_copy(pidx.at[w], iv)

    def chunk(g, carry):
        pltpu.async_copy(dm.at[iv.at[pl.ds(g * CH, CH)]], buf, sem).wait()

        def row(r, carry):
            def kk_body(kk, carry):
                accs = list(carry)
                base = kk * 128
                for u in range(8):
                    x = buf[r, pl.ds(base + u * 16, 16)]
                    accs[u] = jnp.minimum(accs[u], x)
                    accs[8 + u] = jnp.maximum(accs[8 + u], x)
                return tuple(accs)

            carry = lax.fori_loop(0, 39, kk_body, carry)
            accs = list(carry)
            x = buf[r, pl.ds(4984, 16)]
            accs[0] = jnp.minimum(accs[0], x)
            accs[8] = jnp.maximum(accs[8], x)
            return tuple(accs)

        return lax.fori_loop(0, CH, row, carry)

    init = tuple([jnp.full((16,), _BIG, _F32)] * 8
                 + [jnp.full((16,), -_BIG, _F32)] * 8)
    accs = lax.fori_loop(0, TPW_PAD // CH, chunk, init)
    mn = accs[0]
    mx = accs[8]
    for u in range(1, 8):
        mn = jnp.minimum(mn, accs[u])
        mx = jnp.maximum(mx, accs[8 + u])
    stg[0, :] = mn
    stg[1, :] = mx
    pltpu.sync_copy(stg.at[0], out_mn.at[w])
    pltpu.sync_copy(stg.at[1], out_mx.at[w])


def _minmax_call(dm, pidx):
    kfn = pl.kernel(
        _minmax_body,
        out_type=(
            jax.ShapeDtypeStruct((NW, 16), _F32),
            jax.ShapeDtypeStruct((NW, 16), _F32),
        ),
        mesh=_MESH,
        scratch_types=(
            pltpu.VMEM((TPW_PAD,), jnp.int32),
            pltpu.VMEM((CH, PN), _F32),
            pltpu.VMEM((2, 16), _F32),
            pltpu.SemaphoreType.DMA,
        ),
    )
    return kfn(dm, pidx)


def _exp_body(dm, pidx, mn_in, mx_in, out, iv, src, dst, mmn, mmx, sem):
    w = _wid()
    pltpu.sync_copy(pidx.at[w], iv)
    pltpu.sync_copy(mn_in, mmn)
    pltpu.sync_copy(mx_in, mmx)
    mn = mmn[0, :]
    mx = mmx[0, :]
    for i in range(1, NW):
        mn = jnp.minimum(mn, mmn[i, :])
        mx = jnp.maximum(mx, mmx[i, :])
    mn_s = jnp.min(mn)
    mx_s = jnp.max(mx)
    nscale = -1.0 / (mx_s - mn_s)

    def do_chunk(g, nrows_out):
        pltpu.async_copy(dm.at[iv.at[pl.ds(g * CH, CH)]], src, sem).wait()

        def row(r, _):
            def kk_body(kk, _):
                base = kk * 128
                for u in range(8):
                    sl = pl.ds(base + u * 16, 16)
                    dst[r, sl] = jnp.exp(src[r, sl] * nscale)
                return 0

            lax.fori_loop(0, 39, kk_body, 0)
            sl = pl.ds(4984, 16)
            dst[r, sl] = jnp.exp(src[r, sl] * nscale)
            return 0

        lax.fori_loop(0, CH, row, 0)
        pltpu.sync_copy(dst.at[pl.ds(0, nrows_out)],
                        out.at[w].at[pl.ds(g * CH, nrows_out)])

    def chunk(g, _):
        do_chunk(g, CH)
        return 0

    lax.fori_loop(0, TPW // CH, chunk, 0)
    do_chunk(TPW // CH, TPW - (TPW // CH) * CH)


def _exp_call(dm, pidx, mn, mx):
    kfn = pl.kernel(
        _exp_body,
        out_type=jax.ShapeDtypeStruct((NW, TPW, PN), _F32),
        mesh=_MESH,
        scratch_types=(
            pltpu.VMEM((TPW_PAD,), jnp.int32),
            pltpu.VMEM((CH, PN), _F32),
            pltpu.VMEM((CH, PN), _F32),
            pltpu.VMEM((NW, 16), _F32),
            pltpu.VMEM((NW, 16), _F32),
            pltpu.SemaphoreType.DMA,
        ),
    )
    return kfn(dm, pidx, mn, mx)


def _build_p() -> np.ndarray:
    p = np.zeros((L * L, L), np.float32)
    for s in range(L):
        for t in range(L):
            p[((s + t) % L) * L + t, s] = 1.0
    return p


_P_NP = _build_p()


def _attn_body(x_ref, xc_ref, w_ref, b_ref, p_ref, out_ref, outc_ref):
    P = p_ref[...]
    iota = lax.broadcasted_iota(jnp.int32, (1, L), 1)

    def sublayer(si, q_in, kv_in):
        Wq = w_ref[si, 0]
        Wk = w_ref[si, 1]
        Wv = w_ref[si, 2]
        Wo = w_ref[si, 3]
        q = jnp.dot(q_in, Wq, preferred_element_type=_F32) + b_ref[si, 0].reshape(1, D)
        k = jnp.dot(kv_in, Wk, preferred_element_type=_F32) + b_ref[si, 1].reshape(1, D)
        v = jnp.dot(kv_in, Wv, preferred_element_type=_F32) + b_ref[si, 2].reshape(1, D)
        q3 = q.reshape(B, L, D)
        k3 = k.reshape(B, L, D)
        v3 = v.reshape(B, L, D)
        G = lax.dot_general(q3, k3, (((2,), (2,)), ((0,), (0,))),
                            preferred_element_type=_F32)
        mv = jnp.dot(G.reshape(B, L * L), P, preferred_element_type=_F32) * (1.0 / D)
        work = jnp.mean(mv, axis=0, keepdims=True)
        cols = []
        idxs = []
        for _ in range(TOPK):
            mval = jnp.max(work)
            idx = jnp.min(jnp.where(work == mval, iota, L))
            idxs.append(idx)
            sel = (iota == idx).astype(_F32)
            cols.append(jnp.sum(mv * sel, axis=1, keepdims=True))
            work = jnp.where(iota == idx, -_BIG, work)
        wmat = jnp.concatenate(cols, axis=1)
        wmax = jnp.max(wmat, axis=1, keepdims=True)
        e = jnp.exp(wmat - wmax)
        prob = e / jnp.sum(e, axis=1, keepdims=True)
        v2 = jnp.concatenate([v3, v3], axis=1)
        agg = None
        for i in range(TOPK):
            rolled = lax.dynamic_slice(v2, (0, idxs[i], 0), (B, L, D))
            term = prob[:, i].reshape(B, 1, 1) * rolled
            agg = term if agg is None else agg + term
        y = jnp.dot(agg.reshape(B * L, D), Wo, preferred_element_type=_F32)
        return y + b_ref[si, 3].reshape(1, D)

    cur = x_ref[...].reshape(B * L, D)
    cur_c = xc_ref[...].reshape(B * L, D)
    for i in range(LAYERS):
        cur = sublayer(4 * i + 0, cur, cur)
        cur_c = sublayer(4 * i + 1, cur_c, cur_c)
        cur = sublayer(4 * i + 2, cur, cur_c)
        cur_c = sublayer(4 * i + 3, cur_c, cur)
    out_ref[...] = cur.reshape(B, L, D)
    outc_ref[...] = cur_c.reshape(B, L, D)


def _attn_call(x, xc, wall, ball):
    return pl.pallas_call(
        _attn_body,
        out_shape=(
            jax.ShapeDtypeStruct((B, L, D), _F32),
            jax.ShapeDtypeStruct((B, L, D), _F32),
        ),
    )(x, xc, wall, ball, jnp.asarray(_P_NP))


def _pad_idx(a):
    a = a.astype(jnp.int32)
    return jnp.concatenate(
        [a, jnp.repeat(a[:, -1:], TPW_PAD - TPW, axis=1)], axis=1)


def kernel(params, distance_matrix, user, poi, cat, lat, lon, tod, dow, unixtime):
    del lat, lon, unixtime
    uidx = _pad_idx(user)
    pidx = _pad_idx(poi)
    cidx = _pad_idx(cat)
    tidx = _pad_idx(tod)
    didx = _pad_idx(dow)

    tables = (
        params["user_emb"], params["poi_emb"], params["cat_emb"],
        params["tod_emb"], params["dow_emb"],
        params["user_embc"], params["tod_embc"], params["dow_embc"],
    )
    inputs, inputs_cat = _emb_call(uidx, pidx, cidx, tidx, didx, tables)

    mn, mx = _minmax_call(distance_matrix, pidx)
    interval = _exp_call(distance_matrix, pidx, mn, mx)

    seq = []
    for i in range(LAYERS):
        seq += [params["poi_attention"][i], params["cat_attention"][i],
                params["cross_poi_attention"][i], params["cross_cat_attention"][i]]
    wall = jnp.stack([jnp.stack([p["Wq"], p["Wk"], p["Wv"], p["Wo"]]) for p in seq])
    ball = jnp.stack([jnp.stack([p["bq"], p["bk"], p["bv"], p["bo"]]) for p in seq])

    outputs, outputs_cat = _attn_call(inputs, inputs_cat, wall, ball)
    return outputs, outputs_cat, interval


# speedup vs baseline: 1.2633x; 1.2633x over previous
import math
import numpy as np
import jax
import jax.numpy as jnp
from jax import lax
from jax.experimental import pallas as pl
from jax.experimental.pallas import tpu as pltpu
from jax.experimental.pallas import tpu_sc as plsc

B, L = 32, 100
D = 128
LAYERS = 2
TOPK = 4
PN = 5000

NC, NS = 2, 16
NW = NC * NS
TPW = L
TPW_PAD = 104
CH = 8

_MESH = plsc.VectorSubcoreMesh(
    core_axis_name="c", subcore_axis_name="s", num_cores=NC, num_subcores=NS
)

_F32 = jnp.float32
_BIG = 3.0e38


def _wid():
    return lax.axis_index("c") * NS + lax.axis_index("s")


def _acc_add(acc, buf):
    def body(j, _):
        for u in range(8):
            sl = pl.ds(u * 16, 16)
            acc[j, sl] = acc[j, sl] + buf[j, sl]
        return 0
    lax.fori_loop(0, TPW, body, 0)


def _emb_body(uidx, pidx, cidx, tidx, didx,
              ue, pe, ce, te, de, uec, tec, dec,
              out_main, out_cat,
              iu, ip, ic, it, idw, acc, buf, sem):
    w = _wid()
    pltpu.sync_copy(uidx.at[w], iu)
    pltpu.sync_copy(pidx.at[w], ip)
    pltpu.sync_copy(cidx.at[w], ic)
    pltpu.sync_copy(tidx.at[w], it)
    pltpu.sync_copy(didx.at[w], idw)

    for first, rest, out in (
        ((ue, iu), ((pe, ip), (te, it), (de, idw)), out_main),
        ((uec, iu), ((ce, ic), (tec, it), (dec, idw)), out_cat),
    ):
        pltpu.async_copy(first[0].at[first[1].at[pl.ds(0, TPW)]], acc, sem).wait()
        for tbl, iv in rest:
            pltpu.async_copy(tbl.at[iv.at[pl.ds(0, TPW)]], buf, sem).wait()
            _acc_add(acc, buf)
        pltpu.sync_copy(acc, out.at[w])


def _emb_call(uidx, pidx, cidx, tidx, didx, tables):
    kfn = pl.kernel(
        _emb_body,
        out_type=(
            jax.ShapeDtypeStruct((NW, TPW, D), _F32),
            jax.ShapeDtypeStruct((NW, TPW, D), _F32),
        ),
        mesh=_MESH,
        scratch_types=(
            pltpu.VMEM((TPW_PAD,), jnp.int32),
            pltpu.VMEM((TPW_PAD,), jnp.int32),
            pltpu.VMEM((TPW_PAD,), jnp.int32),
            pltpu.VMEM((TPW_PAD,), jnp.int32),
            pltpu.VMEM((TPW_PAD,), jnp.int32),
            pltpu.VMEM((TPW, D), _F32),
            pltpu.VMEM((TPW, D), _F32),
            pltpu.SemaphoreType.DMA,
        ),
    )
    return kfn(uidx, pidx, cidx, tidx, didx, *tables)


def _minmax_body(dm, pidx, out_mn, out_mx, iv, buf, stg, sem):
    w = _wid()
    pltpu.sync_copy(pidx.at[w], iv)

    def chunk(g, carry):
        pltpu.async_copy(dm.at[iv.at[pl.ds(g * CH, CH)]], buf, sem).wait()

        def row(r, carry):
            def kk_body(kk, carry):
                accs = list(carry)
                base = kk * 128
                for u in range(8):
                    x = buf[r, pl.ds(base + u * 16, 16)]
                    accs[u] = jnp.minimum(accs[u], x)
                    accs[8 + u] = jnp.maximum(accs[8 + u], x)
                return tuple(accs)

            carry = lax.fori_loop(0, 39, kk_body, carry)
            accs = list(carry)
            x = buf[r, pl.ds(4984, 16)]
            accs[0] = jnp.minimum(accs[0], x)
            accs[8] = jnp.maximum(accs[8], x)
            return tuple(accs)

        return lax.fori_loop(0, CH, row, carry)

    init = tuple([jnp.full((16,), _BIG, _F32)] * 8
                 + [jnp.full((16,), -_BIG, _F32)] * 8)
    accs = lax.fori_loop(0, TPW_PAD // CH, chunk, init)
    mn = accs[0]
    mx = accs[8]
    for u in range(1, 8):
        mn = jnp.minimum(mn, accs[u])
        mx = jnp.maximum(mx, accs[8 + u])
    stg[0, :] = mn
    stg[1, :] = mx
    pltpu.sync_copy(stg.at[0], out_mn.at[w])
    pltpu.sync_copy(stg.at[1], out_mx.at[w])


def _minmax_call(dm, pidx):
    kfn = pl.kernel(
        _minmax_body,
        compiler_params=pltpu.CompilerParams(use_tc_tiling_on_sc=False),
        out_type=(
            jax.ShapeDtypeStruct((NW, 16), _F32),
            jax.ShapeDtypeStruct((NW, 16), _F32),
        ),
        mesh=_MESH,
        scratch_types=(
            pltpu.VMEM((TPW_PAD,), jnp.int32),
            pltpu.VMEM((CH, PN), _F32),
            pltpu.VMEM((2, 16), _F32),
            pltpu.SemaphoreType.DMA,
        ),
    )
    return kfn(dm, pidx)


def _exp_body(dm, pidx, ns_in, out, iv, src, dst, nsb, sem):
    w = _wid()
    pltpu.sync_copy(pidx.at[w], iv)
    pltpu.sync_copy(ns_in, nsb)
    nscale = nsb[...]

    def do_chunk(g, nrows_out):
        pltpu.async_copy(dm.at[iv.at[pl.ds(g * CH, CH)]], src, sem).wait()

        def row(r, _):
            def kk_body(kk, _):
                base = kk * 128
                for u in range(8):
                    sl = pl.ds(base + u * 16, 16)
                    dst[r, sl] = jnp.exp(src[r, sl] * nscale)
                return 0

            lax.fori_loop(0, 39, kk_body, 0)
            sl = pl.ds(4984, 16)
            dst[r, sl] = jnp.exp(src[r, sl] * nscale)
            return 0

        lax.fori_loop(0, CH, row, 0)
        pltpu.sync_copy(dst.at[pl.ds(0, nrows_out)],
                        out.at[w].at[pl.ds(g * CH, nrows_out)])

    def chunk(g, _):
        do_chunk(g, CH)
        return 0

    lax.fori_loop(0, TPW // CH, chunk, 0)
    do_chunk(TPW // CH, TPW - (TPW // CH) * CH)


def _exp_call(dm, pidx, mn, mx):
    nscale = jnp.full((16,), -1.0, _F32) / (jnp.max(mx) - jnp.min(mn))
    kfn = pl.kernel(
        _exp_body,
        compiler_params=pltpu.CompilerParams(use_tc_tiling_on_sc=False),
        out_type=jax.ShapeDtypeStruct((NW, TPW, PN), _F32),
        mesh=_MESH,
        scratch_types=(
            pltpu.VMEM((TPW_PAD,), jnp.int32),
            pltpu.VMEM((CH, PN), _F32),
            pltpu.VMEM((CH, PN), _F32),
            pltpu.VMEM((16,), _F32),
            pltpu.SemaphoreType.DMA,
        ),
    )
    return kfn(dm, pidx, nscale)


def _build_p() -> np.ndarray:
    p = np.zeros((L * L, L), np.float32)
    for s in range(L):
        for t in range(L):
            p[((s + t) % L) * L + t, s] = 1.0
    return p


_P_NP = _build_p()


def _attn_body(x_ref, xc_ref, w_ref, b_ref, p_ref, out_ref, outc_ref):
    P = p_ref[...]
    iota = lax.broadcasted_iota(jnp.int32, (1, L), 1)
    diffmod = jnp.remainder(
        lax.broadcasted_iota(jnp.int32, (L, L), 1)
        - lax.broadcasted_iota(jnp.int32, (L, L), 0) + L,
        L)

    def sublayer(si, q_in, kv_in):
        Wq = w_ref[si, 0]
        Wk = w_ref[si, 1]
        Wv = w_ref[si, 2]
        Wo = w_ref[si, 3]
        q = jnp.dot(q_in, Wq, preferred_element_type=_F32) + b_ref[si, 0].reshape(1, D)
        k = jnp.dot(kv_in, Wk, preferred_element_type=_F32) + b_ref[si, 1].reshape(1, D)
        v = jnp.dot(kv_in, Wv, preferred_element_type=_F32) + b_ref[si, 2].reshape(1, D)
        q3 = q.reshape(B, L, D)
        k3 = k.reshape(B, L, D)
        v3 = v.reshape(B, L, D)
        G = lax.dot_general(q3, k3, (((2,), (2,)), ((0,), (0,))),
                            preferred_element_type=_F32)
        mv = jnp.dot(G.reshape(B, L * L), P, preferred_element_type=_F32) * (1.0 / D)
        work = jnp.mean(mv, axis=0, keepdims=True)
        cols = []
        idxs = []
        for _ in range(TOPK):
            mval = jnp.max(work)
            idx = jnp.min(jnp.where(work == mval, iota, L))
            idxs.append(idx)
            sel = (iota == idx).astype(_F32)
            cols.append(jnp.sum(mv * sel, axis=1, keepdims=True))
            work = jnp.where(iota == idx, -_BIG, work)
        wmat = jnp.concatenate(cols, axis=1)
        wmax = jnp.max(wmat, axis=1, keepdims=True)
        e = jnp.exp(wmat - wmax)
        prob = e / jnp.sum(e, axis=1, keepdims=True)
        mb = None
        for i in range(TOPK):
            mi = (diffmod == idxs[i]).astype(_F32)
            term = prob[:, i].reshape(B, 1, 1) * mi.reshape(1, L, L)
            mb = term if mb is None else mb + term
        agg = lax.dot_general(mb, v3, (((2,), (1,)), ((0,), (0,))),
                              preferred_element_type=_F32)
        y = jnp.dot(agg.reshape(B * L, D), Wo, preferred_element_type=_F32)
        return y + b_ref[si, 3].reshape(1, D)

    cur = x_ref[...].reshape(B * L, D)
    cur_c = xc_ref[...].reshape(B * L, D)
    for i in range(LAYERS):
        cur = sublayer(4 * i + 0, cur, cur)
        cur_c = sublayer(4 * i + 1, cur_c, cur_c)
        cur = sublayer(4 * i + 2, cur, cur_c)
        cur_c = sublayer(4 * i + 3, cur_c, cur)
    out_ref[...] = cur.reshape(B, L, D)
    outc_ref[...] = cur_c.reshape(B, L, D)


def _attn_call(x, xc, wall, ball):
    return pl.pallas_call(
        _attn_body,
        out_shape=(
            jax.ShapeDtypeStruct((B, L, D), _F32),
            jax.ShapeDtypeStruct((B, L, D), _F32),
        ),
    )(x, xc, wall, ball, jnp.asarray(_P_NP))


def _pad_idx(a):
    a = a.astype(jnp.int32)
    return jnp.concatenate(
        [a, jnp.repeat(a[:, -1:], TPW_PAD - TPW, axis=1)], axis=1)


def kernel(params, distance_matrix, user, poi, cat, lat, lon, tod, dow, unixtime):
    del lat, lon, unixtime
    uidx = _pad_idx(user)
    pidx = _pad_idx(poi)
    cidx = _pad_idx(cat)
    tidx = _pad_idx(tod)
    didx = _pad_idx(dow)

    tables = (
        params["user_emb"], params["poi_emb"], params["cat_emb"],
        params["tod_emb"], params["dow_emb"],
        params["user_embc"], params["tod_embc"], params["dow_embc"],
    )
    inputs, inputs_cat = _emb_call(uidx, pidx, cidx, tidx, didx, tables)

    mn, mx = _minmax_call(distance_matrix, pidx)
    interval = _exp_call(distance_matrix, pidx, mn, mx)

    seq = []
    for i in range(LAYERS):
        seq += [params["poi_attention"][i], params["cat_attention"][i],
                params["cross_poi_attention"][i], params["cross_cat_attention"][i]]
    wall = jnp.stack([jnp.stack([p["Wq"], p["Wk"], p["Wv"], p["Wo"]]) for p in seq])
    ball = jnp.stack([jnp.stack([p["bq"], p["bk"], p["bv"], p["bo"]]) for p in seq])

    outputs, outputs_cat = _attn_call(inputs, inputs_cat, wall, ball)
    return outputs, outputs_cat, interval


# speedup vs baseline: 2.8495x; 2.2556x over previous
import math
import numpy as np
import jax
import jax.numpy as jnp
from jax import lax
from jax.experimental import pallas as pl
from jax.experimental.pallas import tpu as pltpu
from jax.experimental.pallas import tpu_sc as plsc

B, L = 32, 100
D = 128
LAYERS = 2
TOPK = 4
PN = 5000

NC, NS = 2, 16
NW = NC * NS
TPW = L
TPW_PAD = 104
CH = 8

_MESH = plsc.VectorSubcoreMesh(
    core_axis_name="c", subcore_axis_name="s", num_cores=NC, num_subcores=NS
)

_F32 = jnp.float32
_BIG = 3.0e38


def _wid():
    return lax.axis_index("c") * NS + lax.axis_index("s")


def _acc_add(acc, buf):
    def body(j, _):
        for u in range(8):
            sl = pl.ds(u * 16, 16)
            acc[j, sl] = acc[j, sl] + buf[j, sl]
        return 0
    lax.fori_loop(0, TPW, body, 0)


def _emb_body(uidx, pidx, cidx, tidx, didx,
              ue, pe, ce, te, de, uec, tec, dec,
              out_main, out_cat,
              iu, ip, ic, it, idw, b0, b1, b2, b3, b4, b5, b6, b7, sem):
    w = _wid()
    pltpu.sync_copy(uidx.at[w], iu)
    pltpu.sync_copy(pidx.at[w], ip)
    pltpu.sync_copy(cidx.at[w], ic)
    pltpu.sync_copy(tidx.at[w], it)
    pltpu.sync_copy(didx.at[w], idw)

    plan = ((ue, iu, b0), (pe, ip, b1), (te, it, b2), (de, idw, b3),
            (uec, iu, b4), (ce, ic, b5), (tec, it, b6), (dec, idw, b7))
    copies = [pltpu.async_copy(tbl.at[iv.at[pl.ds(0, TPW)]], buf, sem)
              for tbl, iv, buf in plan]
    for cp in copies:
        cp.wait()

    def body(j, _):
        for u in range(8):
            sl = pl.ds(u * 16, 16)
            b0[j, sl] = (b0[j, sl] + b1[j, sl]) + (b2[j, sl] + b3[j, sl])
            b4[j, sl] = (b4[j, sl] + b5[j, sl]) + (b6[j, sl] + b7[j, sl])
        return 0
    lax.fori_loop(0, TPW, body, 0)
    pltpu.sync_copy(b0, out_main.at[w])
    pltpu.sync_copy(b4, out_cat.at[w])


def _emb_call(uidx, pidx, cidx, tidx, didx, tables):
    kfn = pl.kernel(
        _emb_body,
        out_type=(
            jax.ShapeDtypeStruct((NW, TPW, D), _F32),
            jax.ShapeDtypeStruct((NW, TPW, D), _F32),
        ),
        mesh=_MESH,
        scratch_types=(
            pltpu.VMEM((TPW_PAD,), jnp.int32),
            pltpu.VMEM((TPW_PAD,), jnp.int32),
            pltpu.VMEM((TPW_PAD,), jnp.int32),
            pltpu.VMEM((TPW_PAD,), jnp.int32),
            pltpu.VMEM((TPW_PAD,), jnp.int32),
        ) + tuple(pltpu.VMEM((TPW, D), _F32) for _ in range(8)) + (
            pltpu.SemaphoreType.DMA,
        ),
    )
    return kfn(uidx, pidx, cidx, tidx, didx, *tables)


PNP = 5120
NCHUNK = TPW_PAD // CH


def _gather_body(dmp, pidx, raw, out_mn, out_mx, iv, bufs, stg, gsems, wsems):
    w = _wid()
    pltpu.sync_copy(pidx.at[w], iv)

    def gather_args(g, b):
        return dmp.at[iv.at[pl.ds(g * CH, CH)]], bufs.at[b], gsems.at[b]

    pltpu.async_copy(*gather_args(0, 0))
    pltpu.async_copy(*gather_args(1, 1))
    accs = [jnp.full((16,), _BIG, _F32)] * 8 + [jnp.full((16,), -_BIG, _F32)] * 8
    for g in range(NCHUNK):
        b = g % 2
        pltpu.make_async_copy(*gather_args(g, b)).wait()
        nrows = CH if g < NCHUNK - 1 else TPW - (NCHUNK - 1) * CH
        wcp = pltpu.async_copy(bufs.at[b].at[pl.ds(0, nrows)],
                               raw.at[w].at[pl.ds(g * CH, nrows)], wsems.at[b])

        def row(r, carry):
            def kk_body(kk, carry):
                a = list(carry)
                base = kk * 128
                for u in range(8):
                    x = bufs[b, r, pl.ds(base + u * 16, 16)]
                    a[u] = jnp.minimum(a[u], x)
                    a[8 + u] = jnp.maximum(a[8 + u], x)
                return tuple(a)
            return lax.fori_loop(0, 40, kk_body, carry)

        accs = list(lax.fori_loop(0, CH, row, tuple(accs)))
        wcp.wait()
        if g + 2 < NCHUNK:
            pltpu.async_copy(*gather_args(g + 2, b))

    mn = accs[0]
    mx = accs[8]
    for u in range(1, 8):
        mn = jnp.minimum(mn, accs[u])
        mx = jnp.maximum(mx, accs[8 + u])
    stg[0, :] = mn
    stg[1, :] = mx
    pltpu.sync_copy(stg.at[0], out_mn.at[w])
    pltpu.sync_copy(stg.at[1], out_mx.at[w])


def _gather_call(dmp, pidx):
    kfn = pl.kernel(
        _gather_body,
        out_type=(
            jax.ShapeDtypeStruct((NW, TPW, PNP), _F32),
            jax.ShapeDtypeStruct((NW, 16), _F32),
            jax.ShapeDtypeStruct((NW, 16), _F32),
        ),
        mesh=_MESH,
        scratch_types=(
            pltpu.VMEM((TPW_PAD,), jnp.int32),
            pltpu.VMEM((2, CH, PNP), _F32),
            pltpu.VMEM((2, 16), _F32),
            pltpu.SemaphoreType.DMA((2,)),
            pltpu.SemaphoreType.DMA((2,)),
        ),
    )
    return kfn(dmp, pidx)


_EXP_BS = 320


def _exp_tc_body(mn_ref, mx_ref, raw_ref, out_ref):
    scale = -1.0 / (jnp.max(mx_ref[...]) - jnp.min(mn_ref[...]))
    out_ref[...] = jnp.exp(raw_ref[:, :PN] * scale)


def _exp_tc_call(raw, mn, mx):
    return pl.pallas_call(
        _exp_tc_body,
        grid=(B * L // _EXP_BS,),
        in_specs=[
            pl.BlockSpec((NW, 16), lambda i: (0, 0)),
            pl.BlockSpec((NW, 16), lambda i: (0, 0)),
            pl.BlockSpec((_EXP_BS, PNP), lambda i: (i, 0)),
        ],
        out_specs=pl.BlockSpec((_EXP_BS, PN), lambda i: (i, 0)),
        out_shape=jax.ShapeDtypeStruct((B * L, PN), _F32),
    )(mn, mx, raw.reshape(B * L, PNP))


def _build_p() -> np.ndarray:
    p = np.zeros((L * L, L), np.float32)
    for s in range(L):
        for t in range(L):
            p[((s + t) % L) * L + t, s] = 1.0
    return p


_P_NP = _build_p()


def _attn_body(x_ref, xc_ref, w_ref, b_ref, p_ref, out_ref, outc_ref):
    P = p_ref[...]
    iota = lax.broadcasted_iota(jnp.int32, (1, L), 1)
    diffmod = jnp.remainder(
        lax.broadcasted_iota(jnp.int32, (L, L), 1)
        - lax.broadcasted_iota(jnp.int32, (L, L), 0) + L,
        L)

    def sublayer(si, q_in, kv_in):
        Wq = w_ref[si, 0]
        Wk = w_ref[si, 1]
        Wv = w_ref[si, 2]
        Wo = w_ref[si, 3]
        q = jnp.dot(q_in, Wq, preferred_element_type=_F32) + b_ref[si, 0].reshape(1, D)
        k = jnp.dot(kv_in, Wk, preferred_element_type=_F32) + b_ref[si, 1].reshape(1, D)
        v = jnp.dot(kv_in, Wv, preferred_element_type=_F32) + b_ref[si, 2].reshape(1, D)
        q3 = q.reshape(B, L, D)
        k3 = k.reshape(B, L, D)
        v3 = v.reshape(B, L, D)
        G = lax.dot_general(q3, k3, (((2,), (2,)), ((0,), (0,))),
                            preferred_element_type=_F32)
        mv = jnp.dot(G.reshape(B, L * L), P, preferred_element_type=_F32) * (1.0 / D)
        work = jnp.mean(mv, axis=0, keepdims=True)
        cols = []
        idxs = []
        for _ in range(TOPK):
            mval = jnp.max(work)
            idx = jnp.min(jnp.where(work == mval, iota, L))
            idxs.append(idx)
            sel = (iota == idx).astype(_F32)
            cols.append(jnp.sum(mv * sel, axis=1, keepdims=True))
            work = jnp.where(iota == idx, -_BIG, work)
        wmat = jnp.concatenate(cols, axis=1)
        wmax = jnp.max(wmat, axis=1, keepdims=True)
        e = jnp.exp(wmat - wmax)
        prob = e / jnp.sum(e, axis=1, keepdims=True)
        mb = None
        for i in range(TOPK):
            mi = (diffmod == idxs[i]).astype(_F32)
            term = prob[:, i].reshape(B, 1, 1) * mi.reshape(1, L, L)
            mb = term if mb is None else mb + term
        agg = lax.dot_general(mb, v3, (((2,), (1,)), ((0,), (0,))),
                              preferred_element_type=_F32)
        y = jnp.dot(agg.reshape(B * L, D), Wo, preferred_element_type=_F32)
        return y + b_ref[si, 3].reshape(1, D)

    cur = x_ref[...].reshape(B * L, D)
    cur_c = xc_ref[...].reshape(B * L, D)
    for i in range(LAYERS):
        cur = sublayer(4 * i + 0, cur, cur)
        cur_c = sublayer(4 * i + 1, cur_c, cur_c)
        cur = sublayer(4 * i + 2, cur, cur_c)
        cur_c = sublayer(4 * i + 3, cur_c, cur)
    out_ref[...] = cur.reshape(B, L, D)
    outc_ref[...] = cur_c.reshape(B, L, D)


def _attn_call(x, xc, wall, ball):
    return pl.pallas_call(
        _attn_body,
        out_shape=(
            jax.ShapeDtypeStruct((B, L, D), _F32),
            jax.ShapeDtypeStruct((B, L, D), _F32),
        ),
    )(x, xc, wall, ball, jnp.asarray(_P_NP))


def _pad_idx(a):
    a = a.astype(jnp.int32)
    return jnp.concatenate(
        [a, jnp.repeat(a[:, -1:], TPW_PAD - TPW, axis=1)], axis=1)


def kernel(params, distance_matrix, user, poi, cat, lat, lon, tod, dow, unixtime):
    del lat, lon, unixtime
    uidx = _pad_idx(user)
    pidx = _pad_idx(poi)
    cidx = _pad_idx(cat)
    tidx = _pad_idx(tod)
    didx = _pad_idx(dow)

    tables = (
        params["user_emb"], params["poi_emb"], params["cat_emb"],
        params["tod_emb"], params["dow_emb"],
        params["user_embc"], params["tod_embc"], params["dow_embc"],
    )
    inputs, inputs_cat = _emb_call(uidx, pidx, cidx, tidx, didx, tables)

    dmp = jnp.pad(distance_matrix, ((0, 0), (0, PNP - PN)), mode="edge")
    raw, mn, mx = _gather_call(dmp, pidx)
    interval = _exp_tc_call(raw, mn, mx).reshape(B, L, PN)

    seq = []
    for i in range(LAYERS):
        seq += [params["poi_attention"][i], params["cat_attention"][i],
                params["cross_poi_attention"][i], params["cross_cat_attention"][i]]
    wall = jnp.stack([jnp.stack([p["Wq"], p["Wk"], p["Wv"], p["Wo"]]) for p in seq])
    ball = jnp.stack([jnp.stack([p["bq"], p["bk"], p["bv"], p["bo"]]) for p in seq])

    outputs, outputs_cat = _attn_call(inputs, inputs_cat, wall, ball)
    return outputs, outputs_cat, interval


# speedup vs baseline: 4.2849x; 1.5038x over previous
import math
import numpy as np
import jax
import jax.numpy as jnp
from jax import lax
from jax.experimental import pallas as pl
from jax.experimental.pallas import tpu as pltpu
from jax.experimental.pallas import tpu_sc as plsc

B, L = 32, 100
D = 128
LAYERS = 2
TOPK = 4
PN = 5000

NC, NS = 2, 16
NW = NC * NS
TPW = L
TPW_PAD = 104
CH = 8

_MESH = plsc.VectorSubcoreMesh(
    core_axis_name="c", subcore_axis_name="s", num_cores=NC, num_subcores=NS
)

_F32 = jnp.float32
_BIG = 3.0e38


def _wid():
    return lax.axis_index("c") * NS + lax.axis_index("s")


def _acc_add(acc, buf):
    def body(j, _):
        for u in range(8):
            sl = pl.ds(u * 16, 16)
            acc[j, sl] = acc[j, sl] + buf[j, sl]
        return 0
    lax.fori_loop(0, TPW, body, 0)


def _emb_body(uidx, pidx, cidx, tidx, didx,
              ue, pe, ce, te, de, uec, tec, dec,
              out_main, out_cat,
              iu, ip, ic, it, idw, b0, b1, b2, b3, b4, b5, b6, b7, sem):
    w = _wid()
    pltpu.sync_copy(uidx.at[w], iu)
    pltpu.sync_copy(pidx.at[w], ip)
    pltpu.sync_copy(cidx.at[w], ic)
    pltpu.sync_copy(tidx.at[w], it)
    pltpu.sync_copy(didx.at[w], idw)

    plan = ((ue, iu, b0), (pe, ip, b1), (te, it, b2), (de, idw, b3),
            (uec, iu, b4), (ce, ic, b5), (tec, it, b6), (dec, idw, b7))
    copies = [pltpu.async_copy(tbl.at[iv.at[pl.ds(0, TPW)]], buf, sem)
              for tbl, iv, buf in plan]
    for cp in copies:
        cp.wait()

    def body(j, _):
        for u in range(8):
            sl = pl.ds(u * 16, 16)
            b0[j, sl] = (b0[j, sl] + b1[j, sl]) + (b2[j, sl] + b3[j, sl])
            b4[j, sl] = (b4[j, sl] + b5[j, sl]) + (b6[j, sl] + b7[j, sl])
        return 0
    lax.fori_loop(0, TPW, body, 0)
    pltpu.sync_copy(b0, out_main.at[w])
    pltpu.sync_copy(b4, out_cat.at[w])


def _emb_call(uidx, pidx, cidx, tidx, didx, tables):
    kfn = pl.kernel(
        _emb_body,
        out_type=(
            jax.ShapeDtypeStruct((NW, TPW, D), _F32),
            jax.ShapeDtypeStruct((NW, TPW, D), _F32),
        ),
        mesh=_MESH,
        scratch_types=(
            pltpu.VMEM((TPW_PAD,), jnp.int32),
            pltpu.VMEM((TPW_PAD,), jnp.int32),
            pltpu.VMEM((TPW_PAD,), jnp.int32),
            pltpu.VMEM((TPW_PAD,), jnp.int32),
            pltpu.VMEM((TPW_PAD,), jnp.int32),
        ) + tuple(pltpu.VMEM((TPW, D), _F32) for _ in range(8)) + (
            pltpu.SemaphoreType.DMA,
        ),
    )
    return kfn(uidx, pidx, cidx, tidx, didx, *tables)


PNP = 5120
NCHUNK = TPW_PAD // CH


def _gather_body(dmp, pidx, raw, out_mn, out_mx, iv, bufs, stg, gsems, wsems):
    w = _wid()
    pltpu.sync_copy(pidx.at[w], iv)

    def gather_args(g, b):
        return dmp.at[iv.at[pl.ds(g * CH, CH)]], bufs.at[b], gsems.at[b]

    pltpu.async_copy(*gather_args(0, 0))
    pltpu.async_copy(*gather_args(1, 1))
    accs = [jnp.full((16,), _BIG, _F32)] * 8 + [jnp.full((16,), -_BIG, _F32)] * 8
    for g in range(NCHUNK):
        b = g % 2
        pltpu.make_async_copy(*gather_args(g, b)).wait()
        nrows = CH if g < NCHUNK - 1 else TPW - (NCHUNK - 1) * CH
        wcp = pltpu.async_copy(bufs.at[b].at[pl.ds(0, nrows)],
                               raw.at[w].at[pl.ds(g * CH, nrows)], wsems.at[b])

        def row(r, carry):
            def kk_body(kk, carry):
                a = list(carry)
                base = kk * 128
                for u in range(8):
                    x = bufs[b, r, pl.ds(base + u * 16, 16)]
                    a[u] = jnp.minimum(a[u], x)
                    a[8 + u] = jnp.maximum(a[8 + u], x)
                return tuple(a)
            return lax.fori_loop(0, 40, kk_body, carry)

        accs = list(lax.fori_loop(0, CH, row, tuple(accs)))
        wcp.wait()
        if g + 2 < NCHUNK:
            pltpu.async_copy(*gather_args(g + 2, b))

    mn = accs[0]
    mx = accs[8]
    for u in range(1, 8):
        mn = jnp.minimum(mn, accs[u])
        mx = jnp.maximum(mx, accs[8 + u])
    stg[0, :] = mn
    stg[1, :] = mx
    pltpu.sync_copy(stg.at[0], out_mn.at[w])
    pltpu.sync_copy(stg.at[1], out_mx.at[w])


def _gather_call(dmp, pidx):
    kfn = pl.kernel(
        _gather_body,
        out_type=(
            jax.ShapeDtypeStruct((NW, TPW, PNP), _F32),
            jax.ShapeDtypeStruct((NW, 16), _F32),
            jax.ShapeDtypeStruct((NW, 16), _F32),
        ),
        mesh=_MESH,
        scratch_types=(
            pltpu.VMEM((TPW_PAD,), jnp.int32),
            pltpu.VMEM((2, CH, PNP), _F32),
            pltpu.VMEM((2, 16), _F32),
            pltpu.SemaphoreType.DMA((2,)),
            pltpu.SemaphoreType.DMA((2,)),
        ),
    )
    return kfn(dmp, pidx)


_EXP_BB = 4


def _exp_tc_body(mn_ref, mx_ref, raw_ref, out_ref):
    scale = -1.0 / (jnp.max(mx_ref[...]) - jnp.min(mn_ref[...]))
    out_ref[...] = jnp.exp(raw_ref[:, :, :PN] * scale)


def _exp_tc_call(raw, mn, mx):
    return pl.pallas_call(
        _exp_tc_body,
        grid=(NW // _EXP_BB,),
        in_specs=[
            pl.BlockSpec((NW, 16), lambda i: (0, 0)),
            pl.BlockSpec((NW, 16), lambda i: (0, 0)),
            pl.BlockSpec((_EXP_BB, TPW, PNP), lambda i: (i, 0, 0)),
        ],
        out_specs=pl.BlockSpec((_EXP_BB, TPW, PN), lambda i: (i, 0, 0)),
        out_shape=jax.ShapeDtypeStruct((NW, TPW, PN), _F32),
    )(mn, mx, raw)


def _build_p() -> np.ndarray:
    p = np.zeros((L * L, L), np.float32)
    for s in range(L):
        for t in range(L):
            p[((s + t) % L) * L + t, s] = 1.0
    return p


_P_NP = _build_p()


def _attn_body(x_ref, xc_ref, w_ref, b_ref, p_ref, out_ref, outc_ref):
    P = p_ref[...]
    iota = lax.broadcasted_iota(jnp.int32, (1, L), 1)
    diffmod = jnp.remainder(
        lax.broadcasted_iota(jnp.int32, (L, L), 1)
        - lax.broadcasted_iota(jnp.int32, (L, L), 0) + L,
        L)

    def sublayer(si, q_in, kv_in):
        Wq = w_ref[si, 0]
        Wk = w_ref[si, 1]
        Wv = w_ref[si, 2]
        Wo = w_ref[si, 3]
        q = jnp.dot(q_in, Wq, preferred_element_type=_F32) + b_ref[si, 0].reshape(1, D)
        k = jnp.dot(kv_in, Wk, preferred_element_type=_F32) + b_ref[si, 1].reshape(1, D)
        v = jnp.dot(kv_in, Wv, preferred_element_type=_F32) + b_ref[si, 2].reshape(1, D)
        q3 = q.reshape(B, L, D)
        k3 = k.reshape(B, L, D)
        v3 = v.reshape(B, L, D)
        G = lax.dot_general(q3, k3, (((2,), (2,)), ((0,), (0,))),
                            preferred_element_type=_F32)
        mv = jnp.dot(G.reshape(B, L * L), P, preferred_element_type=_F32) * (1.0 / D)
        work = jnp.mean(mv, axis=0, keepdims=True)
        cols = []
        idxs = []
        for _ in range(TOPK):
            mval = jnp.max(work)
            idx = jnp.min(jnp.where(work == mval, iota, L))
            idxs.append(idx)
            sel = (iota == idx).astype(_F32)
            cols.append(jnp.sum(mv * sel, axis=1, keepdims=True))
            work = jnp.where(iota == idx, -_BIG, work)
        wmat = jnp.concatenate(cols, axis=1)
        wmax = jnp.max(wmat, axis=1, keepdims=True)
        e = jnp.exp(wmat - wmax)
        prob = e / jnp.sum(e, axis=1, keepdims=True)
        mb = None
        for i in range(TOPK):
            mi = (diffmod == idxs[i]).astype(_F32)
            term = prob[:, i].reshape(B, 1, 1) * mi.reshape(1, L, L)
            mb = term if mb is None else mb + term
        agg = lax.dot_general(mb, v3, (((2,), (1,)), ((0,), (0,))),
                              preferred_element_type=_F32)
        y = jnp.dot(agg.reshape(B * L, D), Wo, preferred_element_type=_F32)
        return y + b_ref[si, 3].reshape(1, D)

    cur = x_ref[...].reshape(B * L, D)
    cur_c = xc_ref[...].reshape(B * L, D)
    for i in range(LAYERS):
        cur = sublayer(4 * i + 0, cur, cur)
        cur_c = sublayer(4 * i + 1, cur_c, cur_c)
        cur = sublayer(4 * i + 2, cur, cur_c)
        cur_c = sublayer(4 * i + 3, cur_c, cur)
    out_ref[...] = cur.reshape(B, L, D)
    outc_ref[...] = cur_c.reshape(B, L, D)


def _attn_call(x, xc, wall, ball):
    return pl.pallas_call(
        _attn_body,
        out_shape=(
            jax.ShapeDtypeStruct((B, L, D), _F32),
            jax.ShapeDtypeStruct((B, L, D), _F32),
        ),
    )(x, xc, wall, ball, jnp.asarray(_P_NP))


def _pad_idx(a):
    a = a.astype(jnp.int32)
    return jnp.concatenate(
        [a, jnp.repeat(a[:, -1:], TPW_PAD - TPW, axis=1)], axis=1)


def kernel(params, distance_matrix, user, poi, cat, lat, lon, tod, dow, unixtime):
    del lat, lon, unixtime
    uidx = _pad_idx(user)
    pidx = _pad_idx(poi)
    cidx = _pad_idx(cat)
    tidx = _pad_idx(tod)
    didx = _pad_idx(dow)

    tables = (
        params["user_emb"], params["poi_emb"], params["cat_emb"],
        params["tod_emb"], params["dow_emb"],
        params["user_embc"], params["tod_embc"], params["dow_embc"],
    )
    inputs, inputs_cat = _emb_call(uidx, pidx, cidx, tidx, didx, tables)

    dmp = jnp.pad(distance_matrix, ((0, 0), (0, PNP - PN)), mode="edge")
    raw, mn, mx = _gather_call(dmp, pidx)
    interval = _exp_tc_call(raw, mn, mx)

    seq = []
    for i in range(LAYERS):
        seq += [params["poi_attention"][i], params["cat_attention"][i],
                params["cross_poi_attention"][i], params["cross_cat_attention"][i]]
    wall = jnp.stack([jnp.stack([p["Wq"], p["Wk"], p["Wv"], p["Wo"]]) for p in seq])
    ball = jnp.stack([jnp.stack([p["bq"], p["bk"], p["bv"], p["bo"]]) for p in seq])

    outputs, outputs_cat = _attn_call(inputs, inputs_cat, wall, ball)
    return outputs, outputs_cat, interval


# speedup vs baseline: 7.8947x; 1.8424x over previous
import math
import numpy as np
import jax
import jax.numpy as jnp
from jax import lax
from jax.experimental import pallas as pl
from jax.experimental.pallas import tpu as pltpu
from jax.experimental.pallas import tpu_sc as plsc

B, L = 32, 100
D = 128
LAYERS = 2
TOPK = 4
PN = 5000

NC, NS = 2, 16
NW = NC * NS
TPW = L
TPW_PAD = 104
CH = 8

_MESH = plsc.VectorSubcoreMesh(
    core_axis_name="c", subcore_axis_name="s", num_cores=NC, num_subcores=NS
)

_F32 = jnp.float32
_BIG = 3.0e38


def _wid():
    return lax.axis_index("c") * NS + lax.axis_index("s")


def _acc_add(acc, buf):
    def body(j, _):
        for u in range(8):
            sl = pl.ds(u * 16, 16)
            acc[j, sl] = acc[j, sl] + buf[j, sl]
        return 0
    lax.fori_loop(0, TPW, body, 0)


def _emb_body(uidx, pidx, cidx, tdidx,
              ue, pe, ce, td, uec, tdc,
              out_main, out_cat,
              iu, ip, ic, itd, b0, b1, b2, b3, b4, b5, sem):
    w = _wid()
    pltpu.sync_copy(uidx.at[w], iu)
    pltpu.sync_copy(pidx.at[w], ip)
    pltpu.sync_copy(cidx.at[w], ic)
    pltpu.sync_copy(tdidx.at[w], itd)

    plan = ((ue, iu, b0), (pe, ip, b1), (td, itd, b2),
            (uec, iu, b3), (ce, ic, b4), (tdc, itd, b5))
    copies = [pltpu.async_copy(tbl.at[iv.at[pl.ds(0, TPW)]], buf, sem)
              for tbl, iv, buf in plan]
    for cp in copies:
        cp.wait()

    def body(j, _):
        for u in range(8):
            sl = pl.ds(u * 16, 16)
            b0[j, sl] = b0[j, sl] + (b1[j, sl] + b2[j, sl])
            b3[j, sl] = b3[j, sl] + (b4[j, sl] + b5[j, sl])
        return 0
    lax.fori_loop(0, TPW, body, 0)
    pltpu.sync_copy(b0, out_main.at[w])
    pltpu.sync_copy(b3, out_cat.at[w])


def _emb_call(uidx, pidx, cidx, tdidx, tables):
    kfn = pl.kernel(
        _emb_body,
        out_type=(
            jax.ShapeDtypeStruct((NW, TPW, D), _F32),
            jax.ShapeDtypeStruct((NW, TPW, D), _F32),
        ),
        mesh=_MESH,
        scratch_types=(
            pltpu.VMEM((TPW_PAD,), jnp.int32),
            pltpu.VMEM((TPW_PAD,), jnp.int32),
            pltpu.VMEM((TPW_PAD,), jnp.int32),
            pltpu.VMEM((TPW_PAD,), jnp.int32),
        ) + tuple(pltpu.VMEM((TPW, D), _F32) for _ in range(6)) + (
            pltpu.SemaphoreType.DMA,
        ),
    )
    return kfn(uidx, pidx, cidx, tdidx, *tables)


CPN = 4992
NCHUNK = TPW_PAD // CH


def _gather_body(dm, pidx, raw, out_mn, out_mx, iv, bufs, stg, gsems, wsems):
    w = _wid()
    pltpu.sync_copy(pidx.at[w], iv)
    dms = dm.at[:, pl.ds(0, CPN)]

    def gather_args(g, b):
        return dms.at[iv.at[pl.ds(g * CH, CH)]], bufs.at[b], gsems.at[b]

    pltpu.async_copy(*gather_args(0, 0))
    pltpu.async_copy(*gather_args(1, 1))
    accs = [jnp.full((16,), _BIG, _F32)] * 8 + [jnp.full((16,), -_BIG, _F32)] * 8
    for g in range(NCHUNK):
        b = g % 2
        pltpu.make_async_copy(*gather_args(g, b)).wait()
        nrows = CH if g < NCHUNK - 1 else TPW - (NCHUNK - 1) * CH
        wcp = pltpu.async_copy(bufs.at[b].at[pl.ds(0, nrows)],
                               raw.at[pl.ds(g * CH, nrows), w], wsems.at[b])

        def row(r, carry):
            def kk_body(kk, carry):
                a = list(carry)
                base = kk * 128
                for u in range(8):
                    x = bufs[b, r, pl.ds(base + u * 16, 16)]
                    a[u] = jnp.minimum(a[u], x)
                    a[8 + u] = jnp.maximum(a[8 + u], x)
                return tuple(a)
            return lax.fori_loop(0, 39, kk_body, carry)

        accs = list(lax.fori_loop(0, CH, row, tuple(accs)))
        wcp.wait()
        if g + 2 < NCHUNK:
            pltpu.async_copy(*gather_args(g + 2, b))

    mn = accs[0]
    mx = accs[8]
    for u in range(1, 8):
        mn = jnp.minimum(mn, accs[u])
        mx = jnp.maximum(mx, accs[8 + u])
    stg[0, :] = mn
    stg[1, :] = mx
    pltpu.sync_copy(stg.at[0], out_mn.at[w])
    pltpu.sync_copy(stg.at[1], out_mx.at[w])


def _gather_call(dm, pidx):
    kfn = pl.kernel(
        _gather_body,
        out_type=(
            jax.ShapeDtypeStruct((TPW, NW, CPN), _F32),
            jax.ShapeDtypeStruct((NW, 16), _F32),
            jax.ShapeDtypeStruct((NW, 16), _F32),
        ),
        mesh=_MESH,
        scratch_types=(
            pltpu.VMEM((TPW_PAD,), jnp.int32),
            pltpu.VMEM((2, CH, CPN), _F32),
            pltpu.VMEM((2, 16), _F32),
            pltpu.SemaphoreType.DMA((2,)),
            pltpu.SemaphoreType.DMA((2,)),
        ),
    )
    return kfn(dm, pidx)


_EXP_BL = 10


def _exp_tc_body(mn_ref, mx_ref, tail_all_ref, raw_ref, tail_ref, out_ref):
    mn = jnp.minimum(jnp.min(mn_ref[...]), jnp.min(tail_all_ref[...]))
    mx = jnp.maximum(jnp.max(mx_ref[...]), jnp.max(tail_all_ref[...]))
    scale = -1.0 / (mx - mn)
    out_ref[:, :, :CPN] = jnp.exp(raw_ref[...] * scale)
    out_ref[:, :, CPN:] = jnp.exp(tail_ref[...] * scale)


def _exp_tc_call(raw, tail, mn, mx):
    return pl.pallas_call(
        _exp_tc_body,
        grid=(TPW // _EXP_BL,),
        in_specs=[
            pl.BlockSpec((NW, 16), lambda i: (0, 0)),
            pl.BlockSpec((NW, 16), lambda i: (0, 0)),
            pl.BlockSpec((TPW, NW, PN - CPN), lambda i: (0, 0, 0)),
            pl.BlockSpec((_EXP_BL, NW, CPN), lambda i: (i, 0, 0)),
            pl.BlockSpec((_EXP_BL, NW, PN - CPN), lambda i: (i, 0, 0)),
        ],
        out_specs=pl.BlockSpec((_EXP_BL, NW, PN), lambda i: (i, 0, 0)),
        out_shape=jax.ShapeDtypeStruct((TPW, NW, PN), _F32),
    )(mn, mx, tail, raw, tail)


def _build_p() -> np.ndarray:
    p = np.zeros((L * L, L), np.float32)
    for s in range(L):
        for t in range(L):
            p[((s + t) % L) * L + t, s] = 1.0
    return p


_P_NP = _build_p()


def _attn_body(x_ref, xc_ref, w_ref, b_ref, p_ref, out_ref, outc_ref):
    P = p_ref[...]
    iota = lax.broadcasted_iota(jnp.int32, (1, L), 1)
    diffmod = jnp.remainder(
        lax.broadcasted_iota(jnp.int32, (L, L), 1)
        - lax.broadcasted_iota(jnp.int32, (L, L), 0) + L,
        L)

    def sublayer(si, q_in, kv_in):
        Wq = w_ref[si, 0]
        Wk = w_ref[si, 1]
        Wv = w_ref[si, 2]
        Wo = w_ref[si, 3]
        q = jnp.dot(q_in, Wq, preferred_element_type=_F32) + b_ref[si, 0].reshape(1, D)
        k = jnp.dot(kv_in, Wk, preferred_element_type=_F32) + b_ref[si, 1].reshape(1, D)
        v = jnp.dot(kv_in, Wv, preferred_element_type=_F32) + b_ref[si, 2].reshape(1, D)
        q3 = q.reshape(B, L, D)
        k3 = k.reshape(B, L, D)
        v3 = v.reshape(B, L, D)
        G = lax.dot_general(q3, k3, (((2,), (2,)), ((0,), (0,))),
                            preferred_element_type=_F32)
        mv = jnp.dot(G.reshape(B, L * L), P, preferred_element_type=_F32) * (1.0 / D)
        work = jnp.mean(mv, axis=0, keepdims=True)
        cols = []
        idxs = []
        for _ in range(TOPK):
            mval = jnp.max(work)
            idx = jnp.min(jnp.where(work == mval, iota, L))
            idxs.append(idx)
            sel = (iota == idx).astype(_F32)
            cols.append(jnp.sum(mv * sel, axis=1, keepdims=True))
            work = jnp.where(iota == idx, -_BIG, work)
        wmat = jnp.concatenate(cols, axis=1)
        wmax = jnp.max(wmat, axis=1, keepdims=True)
        e = jnp.exp(wmat - wmax)
        prob = e / jnp.sum(e, axis=1, keepdims=True)
        mb = None
        for i in range(TOPK):
            mi = (diffmod == idxs[i]).astype(_F32)
            term = prob[:, i].reshape(B, 1, 1) * mi.reshape(1, L, L)
            mb = term if mb is None else mb + term
        agg = lax.dot_general(mb, v3, (((2,), (1,)), ((0,), (0,))),
                              preferred_element_type=_F32)
        y = jnp.dot(agg.reshape(B * L, D), Wo, preferred_element_type=_F32)
        return y + b_ref[si, 3].reshape(1, D)

    cur = x_ref[...].reshape(B * L, D)
    cur_c = xc_ref[...].reshape(B * L, D)
    for i in range(LAYERS):
        cur = sublayer(4 * i + 0, cur, cur)
        cur_c = sublayer(4 * i + 1, cur_c, cur_c)
        cur = sublayer(4 * i + 2, cur, cur_c)
        cur_c = sublayer(4 * i + 3, cur_c, cur)
    out_ref[...] = cur.reshape(B, L, D)
    outc_ref[...] = cur_c.reshape(B, L, D)


def _attn_call(x, xc, wall, ball):
    return pl.pallas_call(
        _attn_body,
        out_shape=(
            jax.ShapeDtypeStruct((B, L, D), _F32),
            jax.ShapeDtypeStruct((B, L, D), _F32),
        ),
    )(x, xc, wall, ball, jnp.asarray(_P_NP))


def _pad_idx(a):
    a = a.astype(jnp.int32)
    return jnp.concatenate(
        [a, jnp.repeat(a[:, -1:], TPW_PAD - TPW, axis=1)], axis=1)


def kernel(params, distance_matrix, user, poi, cat, lat, lon, tod, dow, unixtime):
    del lat, lon, unixtime
    uidx = _pad_idx(user)
    pidx = _pad_idx(poi)
    cidx = _pad_idx(cat)
    tdidx = _pad_idx(tod * 7 + dow)

    td_tab = (params["tod_emb"][:, None, :] + params["dow_emb"][None, :, :]
              ).reshape(24 * 7, D)
    tdc_tab = (params["tod_embc"][:, None, :] + params["dow_embc"][None, :, :]
               ).reshape(24 * 7, D)
    tables = (
        params["user_emb"], params["poi_emb"], params["cat_emb"],
        td_tab, params["user_embc"], tdc_tab,
    )
    inputs, inputs_cat = _emb_call(uidx, pidx, cidx, tdidx, tables)

    poi_t = poi.T.reshape(-1).astype(jnp.int32)
    tail = jnp.take(distance_matrix[:, CPN:], poi_t, axis=0).reshape(L, B, PN - CPN)

    raw, mn, mx = _gather_call(distance_matrix, pidx)
    interval = _exp_tc_call(raw, tail, mn, mx).transpose(1, 0, 2)

    seq = []
    for i in range(LAYERS):
        seq += [params["poi_attention"][i], params["cat_attention"][i],
                params["cross_poi_attention"][i], params["cross_cat_attention"][i]]
    wall = jnp.stack([jnp.stack([p["Wq"], p["Wk"], p["Wv"], p["Wo"]]) for p in seq])
    ball = jnp.stack([jnp.stack([p["bq"], p["bk"], p["bv"], p["bo"]]) for p in seq])

    outputs, outputs_cat = _attn_call(inputs, inputs_cat, wall, ball)
    return outputs, outputs_cat, interval
